# baseline (speedup 1.0000x reference)
import jax
import jax.numpy as jnp
from jax import lax
from jax.experimental import pallas as pl
from jax.experimental.pallas import tpu as pltpu

_K = 8
_UNROLL = 8


_SORT8_NET = (
    (0, 2), (1, 3), (4, 6), (5, 7),
    (0, 4), (1, 5), (2, 6), (3, 7),
    (0, 1), (2, 3), (4, 5), (6, 7),
    (2, 4), (3, 5),
    (1, 4), (3, 6),
    (1, 2), (3, 4), (5, 6),
)


def _sort8_desc(vs):
    vs = list(vs)
    for i, j in _SORT8_NET:
        hi = jnp.maximum(vs[i], vs[j])
        lo = jnp.minimum(vs[i], vs[j])
        vs[i], vs[j] = hi, lo
    return vs


def _merge_top8(l, r):
    return [jnp.maximum(l[j], r[7 - j]) for j in range(8)]


def _bitonic_sort8(m):
    for d in (4, 2, 1):
        nm = list(m)
        for j in range(8):
            if (j % (2 * d)) < d:
                nm[j] = jnp.maximum(m[j], m[j + d])
                nm[j + d] = jnp.minimum(m[j], m[j + d])
        m = nm
    return m


_NB = 4


def _body(x_ref, o_ref):
    nvreg = x_ref.shape[1]
    c = x_ref.shape[3]
    init = jnp.full((8, c), -jnp.inf, jnp.float32)

    ngrp = 4
    for bb in range(_NB):
        def step(i, carry):
            out = []
            for g, tg in enumerate(carry):
                vg = [x_ref[bb, (i * ngrp + g) * _UNROLL + u]
                      for u in range(_UNROLL)]
                sg = _sort8_desc(vg)
                out.append(tuple(_bitonic_sort8(_merge_top8(list(tg), sg))))
            return tuple(out)

        t0 = tuple([init] * 8)
        sets = lax.fori_loop(0, nvreg // (ngrp * _UNROLL), step, (t0,) * ngrp)
        t = list(sets[0])
        for g in range(1, ngrp):
            t = _bitonic_sort8(_merge_top8(t, list(sets[g])))
        for d in (4, 2):
            r = [pltpu.roll(a, d, 0) for a in t]
            t = _bitonic_sort8(_merge_top8(t, r))
        r = [pltpu.roll(a, 1, 0) for a in t]
        m = _merge_top8(t, r)
        s = m[0]
        for j in range(1, 8):
            s = s + m[j]
        s = s * jnp.float32(1.0 / _K)
        o_ref[bb, :, :] = s[0:1, :]


def kernel(x):
    b, s, c = x.shape
    xr = x.reshape(b, s // 8, 8, c)
    out = pl.pallas_call(
        _body,
        grid=(b // _NB,),
        in_specs=[pl.BlockSpec((_NB, s // 8, 8, c), lambda i: (i, 0, 0, 0))],
        out_specs=pl.BlockSpec((_NB, 1, c), lambda i: (i, 0, 0)),
        out_shape=jax.ShapeDtypeStruct((b, 1, c), jnp.float32),
    )(xr)
    return out.reshape(b, c)
